# baseline (speedup 1.0000x reference)
import jax
import jax.numpy as jnp
from jax.experimental import pallas as pl


_BLK = 2048


def _fused_mlp_kernel(x_ref, w1_ref, w2_ref, w3_ref, mask_ref, out_ref):
    m_t = mask_ref[...].T
    c1 = m_t * w1_ref[...]
    c2 = m_t * w2_ref[...]
    c3 = m_t * w3_ref[...]
    h = jnp.maximum(jnp.dot(x_ref[...], c1, preferred_element_type=jnp.float32), 0.0)
    h = jnp.maximum(jnp.dot(h, c2, preferred_element_type=jnp.float32), 0.0)
    out_ref[...] = jnp.dot(h, c3, preferred_element_type=jnp.float32)


def kernel(x, W1, W2, W3, mask):
    B, D = x.shape
    grid = (B // _BLK,)
    return pl.pallas_call(
        _fused_mlp_kernel,
        grid=grid,
        in_specs=[
            pl.BlockSpec((_BLK, D), lambda i: (i, 0)),
            pl.BlockSpec((64, 64), lambda i: (0, 0)),
            pl.BlockSpec((64, 64), lambda i: (0, 0)),
            pl.BlockSpec((1, 64), lambda i: (0, 0)),
            pl.BlockSpec((64, 64), lambda i: (0, 0)),
        ],
        out_specs=pl.BlockSpec((_BLK, 64), lambda i: (i, 0)),
        out_shape=jax.ShapeDtypeStruct((B, 64), jnp.float32),
    )(x, W1, W2, W3, mask)
